# Initial kernel scaffold; baseline (speedup 1.0000x reference)
import jax
import jax.numpy as jnp
from jax.experimental import pallas as pl


def kernel(adj, user_emb, item_emb):
    raise NotImplementedError("write your pallas kernel here")


# speedup vs baseline: 1.0024x; 1.0024x over previous
import jax
import jax.numpy as jnp
from jax.experimental import pallas as pl

ROWS = 100000
EMB = 128
BLOCK = 10000


def _copy_body(u_ref, i_ref, uo_ref, io_ref):
    uo_ref[...] = u_ref[...]
    io_ref[...] = i_ref[...]


def kernel(adj, user_emb, item_emb):
    del adj
    grid = ROWS // BLOCK
    spec = pl.BlockSpec((BLOCK, EMB), lambda n: (n, 0))
    out = pl.pallas_call(
        _copy_body,
        grid=(grid,),
        in_specs=[spec, spec],
        out_specs=[spec, spec],
        out_shape=[
            jax.ShapeDtypeStruct((ROWS, EMB), jnp.float32),
            jax.ShapeDtypeStruct((ROWS, EMB), jnp.float32),
        ],
    )(user_emb, item_emb)
    return (out[0], out[1])
